# baseline (speedup 1.0000x reference)
import functools

import jax
import jax.numpy as jnp
from jax import lax
from jax.experimental import pallas as pl
from jax.experimental.pallas import tpu as pltpu

_B = 16
_N = 20000
_NOBJ = 20
_NCLS = 3
_SUB = 8
_LANE = 2500


def _tree(vals, fn):
    vals = list(vals)
    while len(vals) > 1:
        nxt = [fn(vals[i], vals[i + 1]) for i in range(0, len(vals) - 1, 2)]
        if len(vals) % 2:
            nxt.append(vals[-1])
        vals = nxt
    return vals[0]


def _match_kernel(obj_ref, priors_ref, locs_ref, scores_ref, neg_ref, misc_ref):
    px0 = priors_ref[0]
    py0 = priors_ref[1]
    px1 = priors_ref[2]
    py1 = priors_ref[3]
    parea = priors_ref[4]

    p_idx = (lax.broadcasted_iota(jnp.int32, (_SUB, _LANE), 0) * _LANE
             + lax.broadcasted_iota(jnp.int32, (_SUB, _LANE), 1))

    ious = []
    for j in range(_NOBJ):
        bx0 = obj_ref[0, 0, j]
        by0 = obj_ref[0, 1, j]
        bx1 = obj_ref[0, 2, j]
        by1 = obj_ref[0, 3, j]
        barea = obj_ref[0, 4, j]
        ltx = jnp.maximum(px0, bx0)
        lty = jnp.maximum(py0, by0)
        rbx = jnp.minimum(px1, bx1)
        rby = jnp.minimum(py1, by1)
        w = jnp.maximum(rbx - ltx, 0.0)
        h = jnp.maximum(rby - lty, 0.0)
        inter = w * h
        ious.append(inter / (barea + parea - inter))

    def _comb(a, b):
        take = b[0] > a[0]
        return (jnp.where(take, b[0], a[0]), jnp.where(take, b[1], a[1]))

    best, besti = _tree([(ious[j], j) for j in range(_NOBJ)], _comb)
    besti = besti.astype(jnp.int32)

    pjs = [jnp.min(jnp.where(ious[j] == jnp.max(ious[j]), p_idx, _N))
           for j in range(_NOBJ)]

    jmax = _tree([jnp.where(p_idx == pjs[j], j, -1) for j in range(_NOBJ)],
                 jnp.maximum)
    hit = jmax >= 0
    besti = jnp.where(hit, jmax, besti)
    best = jnp.where(hit, 1.0, best)

    eqs = [besti == j for j in range(_NOBJ)]

    def _gather(row):
        return _tree([jnp.where(eqs[j], obj_ref[0, row, j], 0.0)
                      for j in range(_NOBJ)], lambda a, b: a + b)

    lab = _gather(9)
    bcx = _gather(5)
    bcy = _gather(6)
    bw = _gather(7)
    bh = _gather(8)

    lab = jnp.where(best < 0.5, 0.0, lab)
    posf = (lab > 0.0).astype(jnp.float32)
    n_pos = jnp.sum(posf)

    g0 = (bcx - priors_ref[5]) / priors_ref[7]
    g1 = (bcy - priors_ref[6]) / priors_ref[8]
    g2 = jnp.log(bw / priors_ref[9]) * 5.0
    g3 = jnp.log(bh / priors_ref[10]) * 5.0
    loc_sum = jnp.sum((jnp.abs(locs_ref[0, 0] - g0) + jnp.abs(locs_ref[0, 1] - g1)
                       + jnp.abs(locs_ref[0, 2] - g2) + jnp.abs(locs_ref[0, 3] - g3))
                      * posf)

    t0 = (lab == 1.0).astype(jnp.float32)
    t_bg = (lab == 0.0).astype(jnp.float32)
    t_pair = (lab == 3.0).astype(jnp.float32)
    tgt0 = t_bg
    tgt1 = t0 + t_pair
    tgt2 = (lab == 2.0).astype(jnp.float32) + t_pair
    bce = jnp.zeros((_SUB, _LANE), jnp.float32)
    for c, tgt in enumerate((tgt0, tgt1, tgt2)):
        l = scores_ref[0, c]
        bce = bce + (jnp.maximum(l, 0.0) - l * tgt + jnp.log1p(jnp.exp(-jnp.abs(l))))

    conf_pos = jnp.sum(bce * posf)
    neg_ref[0] = jnp.where(posf > 0.0, 0.0, bce)

    lane = lax.broadcasted_iota(jnp.int32, (1, 128), 1)
    misc_ref[0] = (jnp.where(lane == 0, loc_sum, 0.0)
                   + jnp.where(lane == 1, conf_pos, 0.0)
                   + jnp.where(lane == 2, n_pos, 0.0))


def _topk_kernel(neg_ref, misc_ref, total_ref, conf_ref, loc_ref):
    neg = neg_ref[...]
    bits = lax.bitcast_convert_type(neg, jnp.int32)
    misc = misc_ref[...]
    lane = lax.broadcasted_iota(jnp.int32, misc.shape, 2)
    loc_sum = jnp.sum(jnp.where(lane == 0, misc, 0.0))
    conf_pos = jnp.sum(jnp.where(lane == 1, misc, 0.0))
    nposv = jnp.sum(jnp.where(lane == 2, misc, 0.0), axis=2, keepdims=True)
    n_pos_tot = jnp.sum(nposv)
    k = jnp.minimum(nposv * 3.0, float(_N)).astype(jnp.int32)

    m = jnp.zeros((_B, 1, 1), jnp.int32)
    for bit in range(30, -1, -1):
        cand = m | (1 << bit)
        cnt = jnp.sum((bits >= cand).astype(jnp.int32), axis=(1, 2), keepdims=True)
        m = jnp.where(cnt >= k, cand, m)
    tval = lax.bitcast_convert_type(m, jnp.float32)
    gt = bits > m
    cnt_gt = jnp.sum(gt.astype(jnp.float32), axis=(1, 2), keepdims=True)
    sum_gt = jnp.sum(jnp.where(gt, neg, 0.0), axis=(1, 2), keepdims=True)
    s = sum_gt + (k.astype(jnp.float32) - cnt_gt) * tval
    s = jnp.where(k == 0, 0.0, s)
    hard = jnp.sum(s)

    conf_loss = (hard + conf_pos) / (1e-10 + n_pos_tot)
    loc_loss = loc_sum / jnp.maximum(4.0 * n_pos_tot, 1.0)
    total_ref[0, 0] = conf_loss + loc_loss
    conf_ref[0, 0] = conf_loss
    loc_ref[0, 0] = loc_loss


@functools.partial(jax.jit, static_argnames=())
def kernel(predicted_locs, predicted_scores, boxes, labels, priors_cxcy):
    pcx = priors_cxcy[:, 0]
    pcy = priors_cxcy[:, 1]
    pw = priors_cxcy[:, 2]
    ph = priors_cxcy[:, 3]
    px0 = pcx - pw / 2.0
    py0 = pcy - ph / 2.0
    px1 = pcx + pw / 2.0
    py1 = pcy + ph / 2.0
    parea = (px1 - px0) * (py1 - py0)
    priors_pl = jnp.stack(
        [px0, py0, px1, py1, parea, pcx, pcy, pw / 10.0, ph / 10.0, pw, ph]
    ).reshape(11, _SUB, _LANE)

    bx0 = boxes[:, :, 0]
    by0 = boxes[:, :, 1]
    bx1 = boxes[:, :, 2]
    by1 = boxes[:, :, 3]
    obj = jnp.stack(
        [bx0, by0, bx1, by1,
         (bx1 - bx0) * (by1 - by0),
         (bx0 + bx1) / 2.0, (by0 + by1) / 2.0,
         bx1 - bx0, by1 - by0,
         labels.astype(jnp.float32)],
        axis=1)

    locs_t = predicted_locs.transpose(0, 2, 1).reshape(_B, 4, _SUB, _LANE)
    scores_t = predicted_scores.transpose(0, 2, 1).reshape(_B, _NCLS, _SUB, _LANE)

    neg, misc = pl.pallas_call(
        _match_kernel,
        grid=(_B,),
        in_specs=[
            pl.BlockSpec((1, 10, _NOBJ), lambda b: (b, 0, 0),
                         memory_space=pltpu.SMEM),
            pl.BlockSpec((11, _SUB, _LANE), lambda b: (0, 0, 0)),
            pl.BlockSpec((1, 4, _SUB, _LANE), lambda b: (b, 0, 0, 0)),
            pl.BlockSpec((1, _NCLS, _SUB, _LANE), lambda b: (b, 0, 0, 0)),
        ],
        out_specs=[
            pl.BlockSpec((1, _SUB, _LANE), lambda b: (b, 0, 0)),
            pl.BlockSpec((1, 1, 128), lambda b: (b, 0, 0)),
        ],
        out_shape=[
            jax.ShapeDtypeStruct((_B, _SUB, _LANE), jnp.float32),
            jax.ShapeDtypeStruct((_B, 1, 128), jnp.float32),
        ],
    )(obj, priors_pl, locs_t, scores_t)

    total, conf, loc = pl.pallas_call(
        _topk_kernel,
        in_specs=[
            pl.BlockSpec((_B, _SUB, _LANE), lambda: (0, 0, 0)),
            pl.BlockSpec((_B, 1, 128), lambda: (0, 0, 0)),
        ],
        out_specs=[
            pl.BlockSpec((1, 1), lambda: (0, 0), memory_space=pltpu.SMEM),
            pl.BlockSpec((1, 1), lambda: (0, 0), memory_space=pltpu.SMEM),
            pl.BlockSpec((1, 1), lambda: (0, 0), memory_space=pltpu.SMEM),
        ],
        out_shape=[
            jax.ShapeDtypeStruct((1, 1), jnp.float32),
            jax.ShapeDtypeStruct((1, 1), jnp.float32),
            jax.ShapeDtypeStruct((1, 1), jnp.float32),
        ],
    )(neg, misc)

    n_positives = misc[:, 0, 2].astype(jnp.int32)
    return total[0, 0], conf[0, 0], loc[0, 0], n_positives


# speedup vs baseline: 1.1472x; 1.1472x over previous
import functools

import jax
import jax.numpy as jnp
from jax import lax
from jax.experimental import pallas as pl
from jax.experimental.pallas import tpu as pltpu

_B = 16
_N = 20000
_NOBJ = 20
_NCLS = 3
_SUB = 8
_LANE = 2500


def _fused_kernel(obj_ref, packs_ref, priors_ref, locs_ref, scores_ref,
                  total_ref, conf_ref, loc_ref, npos_ref,
                  neg_sc, acc_sc):
    b = pl.program_id(0)

    px0 = priors_ref[0]
    py0 = priors_ref[1]
    px1 = priors_ref[2]
    py1 = priors_ref[3]
    parea = priors_ref[4]

    p_idx = (lax.broadcasted_iota(jnp.int32, (_SUB, _LANE), 0) * _LANE
             + lax.broadcasted_iota(jnp.int32, (_SUB, _LANE), 1))

    best = None
    besti = None
    pjs = []
    for j in range(_NOBJ):
        bx0 = obj_ref[0, 0, j]
        by0 = obj_ref[0, 1, j]
        bx1 = obj_ref[0, 2, j]
        by1 = obj_ref[0, 3, j]
        barea = obj_ref[0, 4, j]
        ltx = jnp.maximum(px0, bx0)
        lty = jnp.maximum(py0, by0)
        rbx = jnp.minimum(px1, bx1)
        rby = jnp.minimum(py1, by1)
        w = jnp.maximum(rbx - ltx, 0.0)
        h = jnp.maximum(rby - lty, 0.0)
        inter = w * h
        iou = inter / (barea + parea - inter)
        if j == 0:
            best = iou
            besti = jnp.zeros((_SUB, _LANE), jnp.int32)
        else:
            gt = iou > best
            besti = jnp.where(gt, j, besti)
            best = jnp.where(gt, iou, best)
        pjs.append(jnp.min(jnp.where(iou == jnp.max(iou), p_idx, _N)))

    for j in range(_NOBJ):
        m = p_idx == pjs[j]
        besti = jnp.where(m, j, besti)
        best = jnp.where(m, 1.0, best)

    pa = packs_ref[0, 0, 0]
    pb = packs_ref[0, 0, 1]
    sh = besti + besti
    lab_lo = jnp.right_shift(pa, sh) & 3
    lab_hi = jnp.right_shift(pb, jnp.maximum(sh - 20, 0)) & 3
    lab = jnp.where(besti < 10, lab_lo, lab_hi).astype(jnp.float32)

    bcx = jnp.zeros((_SUB, _LANE), jnp.float32)
    bcy = jnp.zeros((_SUB, _LANE), jnp.float32)
    bw = jnp.zeros((_SUB, _LANE), jnp.float32)
    bh = jnp.zeros((_SUB, _LANE), jnp.float32)
    for j in range(_NOBJ):
        m = besti == j
        bcx = jnp.where(m, obj_ref[0, 5, j], bcx)
        bcy = jnp.where(m, obj_ref[0, 6, j], bcy)
        bw = jnp.where(m, obj_ref[0, 7, j], bw)
        bh = jnp.where(m, obj_ref[0, 8, j], bh)

    lab = jnp.where(best < 0.5, 0.0, lab)
    posf = (lab > 0.0).astype(jnp.float32)
    n_pos = jnp.sum(posf)

    g0 = (bcx - priors_ref[5]) / priors_ref[7]
    g1 = (bcy - priors_ref[6]) / priors_ref[8]
    g2 = jnp.log(bw / priors_ref[9]) * 5.0
    g3 = jnp.log(bh / priors_ref[10]) * 5.0
    loc_sum = jnp.sum((jnp.abs(locs_ref[0, 0] - g0) + jnp.abs(locs_ref[0, 1] - g1)
                       + jnp.abs(locs_ref[0, 2] - g2) + jnp.abs(locs_ref[0, 3] - g3))
                      * posf)

    t_pair = (lab == 3.0).astype(jnp.float32)
    tgt0 = (lab == 0.0).astype(jnp.float32)
    tgt1 = (lab == 1.0).astype(jnp.float32) + t_pair
    tgt2 = (lab == 2.0).astype(jnp.float32) + t_pair
    bce = jnp.zeros((_SUB, _LANE), jnp.float32)
    for c, tgt in enumerate((tgt0, tgt1, tgt2)):
        l = scores_ref[0, c]
        bce = bce + (jnp.maximum(l, 0.0) - l * tgt + jnp.log1p(jnp.exp(-jnp.abs(l))))

    conf_pos = jnp.sum(bce * posf)
    neg_sc[b] = jnp.where(posf > 0.0, 0.0, bce)
    npos_ref[b] = n_pos.astype(jnp.int32)

    @pl.when(b == 0)
    def _init():
        acc_sc[0, 0] = loc_sum
        acc_sc[0, 1] = conf_pos

    @pl.when(b > 0)
    def _acc():
        acc_sc[0, 0] = acc_sc[0, 0] + loc_sum
        acc_sc[0, 1] = acc_sc[0, 1] + conf_pos

    @pl.when(b == _B - 1)
    def _epilogue():
        neg = neg_sc[...]
        bits = lax.bitcast_convert_type(neg, jnp.int32)
        img = lax.broadcasted_iota(jnp.int32, (_B, 1, 1), 0)
        nposv = jnp.zeros((_B, 1, 1), jnp.int32)
        for i in range(_B):
            nposv = jnp.where(img == i, npos_ref[i], nposv)
        n_pos_tot = jnp.sum(nposv).astype(jnp.float32)
        k = jnp.minimum(nposv * 3, _N)

        m = jnp.zeros((_B, 1, 1), jnp.int32)
        for bit in range(30, -1, -1):
            cand = m | (1 << bit)
            cnt = jnp.sum((bits >= cand).astype(jnp.int32), axis=(1, 2),
                          keepdims=True)
            m = jnp.where(cnt >= k, cand, m)
        tval = lax.bitcast_convert_type(m, jnp.float32)
        gtm = bits > m
        cnt_gt = jnp.sum(gtm.astype(jnp.float32), axis=(1, 2), keepdims=True)
        sum_gt = jnp.sum(jnp.where(gtm, neg, 0.0), axis=(1, 2), keepdims=True)
        s = sum_gt + (k.astype(jnp.float32) - cnt_gt) * tval
        s = jnp.where(k == 0, 0.0, s)
        hard = jnp.sum(s)

        conf_loss = (hard + acc_sc[0, 1]) / (1e-10 + n_pos_tot)
        loc_loss = acc_sc[0, 0] / jnp.maximum(4.0 * n_pos_tot, 1.0)
        total_ref[0, 0] = conf_loss + loc_loss
        conf_ref[0, 0] = conf_loss
        loc_ref[0, 0] = loc_loss


@functools.partial(jax.jit, static_argnames=())
def kernel(predicted_locs, predicted_scores, boxes, labels, priors_cxcy):
    pcx = priors_cxcy[:, 0]
    pcy = priors_cxcy[:, 1]
    pw = priors_cxcy[:, 2]
    ph = priors_cxcy[:, 3]
    px0 = pcx - pw / 2.0
    py0 = pcy - ph / 2.0
    px1 = pcx + pw / 2.0
    py1 = pcy + ph / 2.0
    parea = (px1 - px0) * (py1 - py0)
    priors_pl = jnp.stack(
        [px0, py0, px1, py1, parea, pcx, pcy, pw / 10.0, ph / 10.0, pw, ph]
    ).reshape(11, _SUB, _LANE)

    bx0 = boxes[:, :, 0]
    by0 = boxes[:, :, 1]
    bx1 = boxes[:, :, 2]
    by1 = boxes[:, :, 3]
    obj = jnp.stack(
        [bx0, by0, bx1, by1,
         (bx1 - bx0) * (by1 - by0),
         (bx0 + bx1) / 2.0, (by0 + by1) / 2.0,
         bx1 - bx0, by1 - by0],
        axis=1)

    lab32 = labels.astype(jnp.int32)
    shifts = jnp.arange(10, dtype=jnp.int32) * 2
    pa = jnp.sum(lab32[:, :10] << shifts[None, :], axis=1)
    pb = jnp.sum(lab32[:, 10:] << shifts[None, :], axis=1)
    packs = jnp.stack([pa, pb], axis=1).reshape(_B, 1, 2)

    locs_t = predicted_locs.transpose(0, 2, 1).reshape(_B, 4, _SUB, _LANE)
    scores_t = predicted_scores.transpose(0, 2, 1).reshape(_B, _NCLS, _SUB, _LANE)

    total, conf, loc, npos = pl.pallas_call(
        _fused_kernel,
        grid=(_B,),
        in_specs=[
            pl.BlockSpec((1, 9, _NOBJ), lambda b: (b, 0, 0),
                         memory_space=pltpu.SMEM),
            pl.BlockSpec((1, 1, 2), lambda b: (b, 0, 0),
                         memory_space=pltpu.SMEM),
            pl.BlockSpec((11, _SUB, _LANE), lambda b: (0, 0, 0)),
            pl.BlockSpec((1, 4, _SUB, _LANE), lambda b: (b, 0, 0, 0)),
            pl.BlockSpec((1, _NCLS, _SUB, _LANE), lambda b: (b, 0, 0, 0)),
        ],
        out_specs=[
            pl.BlockSpec((1, 1), lambda b: (0, 0), memory_space=pltpu.SMEM),
            pl.BlockSpec((1, 1), lambda b: (0, 0), memory_space=pltpu.SMEM),
            pl.BlockSpec((1, 1), lambda b: (0, 0), memory_space=pltpu.SMEM),
            pl.BlockSpec((_B,), lambda b: (0,), memory_space=pltpu.SMEM),
        ],
        out_shape=[
            jax.ShapeDtypeStruct((1, 1), jnp.float32),
            jax.ShapeDtypeStruct((1, 1), jnp.float32),
            jax.ShapeDtypeStruct((1, 1), jnp.float32),
            jax.ShapeDtypeStruct((_B,), jnp.int32),
        ],
        scratch_shapes=[
            pltpu.VMEM((_B, _SUB, _LANE), jnp.float32),
            pltpu.SMEM((1, 2), jnp.float32),
        ],
    )(obj, packs, priors_pl, locs_t, scores_t)

    return total[0, 0], conf[0, 0], loc[0, 0], npos


# speedup vs baseline: 1.8120x; 1.5795x over previous
import functools

import jax
import jax.numpy as jnp
from jax import lax
from jax.experimental import pallas as pl
from jax.experimental.pallas import tpu as pltpu

_B = 16
_N = 20000
_NOBJ = 20
_NCLS = 3
_SUB = 8
_LANE = 2500


def _fused_kernel(obj_ref, packs_ref, priors_ref, locs_ref, scores_ref,
                  total_ref, conf_ref, loc_ref, npos_ref,
                  neg_sc, acc_sc, iou_sc):
    b = pl.program_id(0)

    px0 = priors_ref[0]
    py0 = priors_ref[1]
    px1 = priors_ref[2]
    py1 = priors_ref[3]
    parea = priors_ref[4]

    p_idx = (lax.broadcasted_iota(jnp.int32, (_SUB, _LANE), 0) * _LANE
             + lax.broadcasted_iota(jnp.int32, (_SUB, _LANE), 1))

    best = None
    besti = None
    for j in range(_NOBJ):
        bx0 = obj_ref[0, 0, j]
        by0 = obj_ref[0, 1, j]
        bx1 = obj_ref[0, 2, j]
        by1 = obj_ref[0, 3, j]
        barea = obj_ref[0, 4, j]
        ltx = jnp.maximum(px0, bx0)
        lty = jnp.maximum(py0, by0)
        rbx = jnp.minimum(px1, bx1)
        rby = jnp.minimum(py1, by1)
        w = jnp.maximum(rbx - ltx, 0.0)
        h = jnp.maximum(rby - lty, 0.0)
        inter = w * h
        iou = inter / (barea + parea - inter)
        if j == 0:
            best = iou
            besti = jnp.zeros((_SUB, _LANE), jnp.int32)
        else:
            gt = iou > best
            besti = jnp.where(gt, j, besti)
            best = jnp.where(gt, iou, best)
        iou_sc[j] = iou

    iou_all = iou_sc[...]
    mx = jnp.max(iou_all, axis=(1, 2), keepdims=True)
    pidx3 = p_idx[None]
    pj = jnp.min(jnp.where(iou_all == mx, pidx3, _N), axis=(1, 2),
                 keepdims=True)

    jv = lax.broadcasted_iota(jnp.int32, (_NOBJ, 1, 1), 0)
    jmax = jnp.max(jnp.where(pidx3 == pj, jv, -1), axis=0)
    hit = jmax >= 0
    besti = jnp.where(hit, jmax, besti)
    best = jnp.where(hit, 1.0, best)

    pa = packs_ref[0, 0, 0]
    pb = packs_ref[0, 0, 1]
    sh = besti + besti
    lab_lo = jnp.right_shift(pa, sh) & 3
    lab_hi = jnp.right_shift(pb, jnp.maximum(sh - 20, 0)) & 3
    lab = jnp.where(besti < 10, lab_lo, lab_hi).astype(jnp.float32)

    bcx = jnp.zeros((_SUB, _LANE), jnp.float32)
    bcy = jnp.zeros((_SUB, _LANE), jnp.float32)
    bw = jnp.zeros((_SUB, _LANE), jnp.float32)
    bh = jnp.zeros((_SUB, _LANE), jnp.float32)
    for j in range(_NOBJ):
        m = besti == j
        bcx = jnp.where(m, obj_ref[0, 5, j], bcx)
        bcy = jnp.where(m, obj_ref[0, 6, j], bcy)
        bw = jnp.where(m, obj_ref[0, 7, j], bw)
        bh = jnp.where(m, obj_ref[0, 8, j], bh)

    lab = jnp.where(best < 0.5, 0.0, lab)
    posf = (lab > 0.0).astype(jnp.float32)
    n_pos = jnp.sum(posf)

    g0 = (bcx - priors_ref[5]) / priors_ref[7]
    g1 = (bcy - priors_ref[6]) / priors_ref[8]
    g2 = jnp.log(bw / priors_ref[9]) * 5.0
    g3 = jnp.log(bh / priors_ref[10]) * 5.0
    loc_sum = jnp.sum((jnp.abs(locs_ref[0, 0] - g0) + jnp.abs(locs_ref[0, 1] - g1)
                       + jnp.abs(locs_ref[0, 2] - g2) + jnp.abs(locs_ref[0, 3] - g3))
                      * posf)

    t_pair = (lab == 3.0).astype(jnp.float32)
    tgt0 = (lab == 0.0).astype(jnp.float32)
    tgt1 = (lab == 1.0).astype(jnp.float32) + t_pair
    tgt2 = (lab == 2.0).astype(jnp.float32) + t_pair
    bce = jnp.zeros((_SUB, _LANE), jnp.float32)
    for c, tgt in enumerate((tgt0, tgt1, tgt2)):
        l = scores_ref[0, c]
        bce = bce + (jnp.maximum(l, 0.0) - l * tgt + jnp.log1p(jnp.exp(-jnp.abs(l))))

    conf_pos = jnp.sum(bce * posf)
    neg_sc[b] = jnp.where(posf > 0.0, 0.0, bce)
    npos_ref[b] = n_pos.astype(jnp.int32)

    @pl.when(b == 0)
    def _init():
        acc_sc[0, 0] = loc_sum
        acc_sc[0, 1] = conf_pos

    @pl.when(b > 0)
    def _acc():
        acc_sc[0, 0] = acc_sc[0, 0] + loc_sum
        acc_sc[0, 1] = acc_sc[0, 1] + conf_pos

    @pl.when(b == _B - 1)
    def _epilogue():
        neg = neg_sc[...]
        bits = lax.bitcast_convert_type(neg, jnp.int32)
        img = lax.broadcasted_iota(jnp.int32, (_B, 1, 1), 0)
        nposv = jnp.zeros((_B, 1, 1), jnp.int32)
        for i in range(_B):
            nposv = jnp.where(img == i, npos_ref[i], nposv)
        n_pos_tot = jnp.sum(nposv).astype(jnp.float32)
        k = jnp.minimum(nposv * 3, _N)

        m = jnp.zeros((_B, 1, 1), jnp.int32)
        for bit in range(30, -1, -1):
            cand = m | (1 << bit)
            cnt = jnp.sum((bits >= cand).astype(jnp.int32), axis=(1, 2),
                          keepdims=True)
            m = jnp.where(cnt >= k, cand, m)
        tval = lax.bitcast_convert_type(m, jnp.float32)
        gtm = bits > m
        cnt_gt = jnp.sum(gtm.astype(jnp.float32), axis=(1, 2), keepdims=True)
        sum_gt = jnp.sum(jnp.where(gtm, neg, 0.0), axis=(1, 2), keepdims=True)
        s = sum_gt + (k.astype(jnp.float32) - cnt_gt) * tval
        s = jnp.where(k == 0, 0.0, s)
        hard = jnp.sum(s)

        conf_loss = (hard + acc_sc[0, 1]) / (1e-10 + n_pos_tot)
        loc_loss = acc_sc[0, 0] / jnp.maximum(4.0 * n_pos_tot, 1.0)
        total_ref[0, 0] = conf_loss + loc_loss
        conf_ref[0, 0] = conf_loss
        loc_ref[0, 0] = loc_loss


@functools.partial(jax.jit, static_argnames=())
def kernel(predicted_locs, predicted_scores, boxes, labels, priors_cxcy):
    pcx = priors_cxcy[:, 0]
    pcy = priors_cxcy[:, 1]
    pw = priors_cxcy[:, 2]
    ph = priors_cxcy[:, 3]
    px0 = pcx - pw / 2.0
    py0 = pcy - ph / 2.0
    px1 = pcx + pw / 2.0
    py1 = pcy + ph / 2.0
    parea = (px1 - px0) * (py1 - py0)
    priors_pl = jnp.stack(
        [px0, py0, px1, py1, parea, pcx, pcy, pw / 10.0, ph / 10.0, pw, ph]
    ).reshape(11, _SUB, _LANE)

    bx0 = boxes[:, :, 0]
    by0 = boxes[:, :, 1]
    bx1 = boxes[:, :, 2]
    by1 = boxes[:, :, 3]
    obj = jnp.stack(
        [bx0, by0, bx1, by1,
         (bx1 - bx0) * (by1 - by0),
         (bx0 + bx1) / 2.0, (by0 + by1) / 2.0,
         bx1 - bx0, by1 - by0],
        axis=1)

    lab32 = labels.astype(jnp.int32)
    shifts = jnp.arange(10, dtype=jnp.int32) * 2
    pa = jnp.sum(lab32[:, :10] << shifts[None, :], axis=1)
    pb = jnp.sum(lab32[:, 10:] << shifts[None, :], axis=1)
    packs = jnp.stack([pa, pb], axis=1).reshape(_B, 1, 2)

    locs_t = predicted_locs.transpose(0, 2, 1).reshape(_B, 4, _SUB, _LANE)
    scores_t = predicted_scores.transpose(0, 2, 1).reshape(_B, _NCLS, _SUB, _LANE)

    total, conf, loc, npos = pl.pallas_call(
        _fused_kernel,
        grid=(_B,),
        in_specs=[
            pl.BlockSpec((1, 9, _NOBJ), lambda b: (b, 0, 0),
                         memory_space=pltpu.SMEM),
            pl.BlockSpec((1, 1, 2), lambda b: (b, 0, 0),
                         memory_space=pltpu.SMEM),
            pl.BlockSpec((11, _SUB, _LANE), lambda b: (0, 0, 0)),
            pl.BlockSpec((1, 4, _SUB, _LANE), lambda b: (b, 0, 0, 0)),
            pl.BlockSpec((1, _NCLS, _SUB, _LANE), lambda b: (b, 0, 0, 0)),
        ],
        out_specs=[
            pl.BlockSpec((1, 1), lambda b: (0, 0), memory_space=pltpu.SMEM),
            pl.BlockSpec((1, 1), lambda b: (0, 0), memory_space=pltpu.SMEM),
            pl.BlockSpec((1, 1), lambda b: (0, 0), memory_space=pltpu.SMEM),
            pl.BlockSpec((_B,), lambda b: (0,), memory_space=pltpu.SMEM),
        ],
        out_shape=[
            jax.ShapeDtypeStruct((1, 1), jnp.float32),
            jax.ShapeDtypeStruct((1, 1), jnp.float32),
            jax.ShapeDtypeStruct((1, 1), jnp.float32),
            jax.ShapeDtypeStruct((_B,), jnp.int32),
        ],
        scratch_shapes=[
            pltpu.VMEM((_B, _SUB, _LANE), jnp.float32),
            pltpu.SMEM((1, 2), jnp.float32),
            pltpu.VMEM((_NOBJ, _SUB, _LANE), jnp.float32),
        ],
    )(obj, packs, priors_pl, locs_t, scores_t)

    return total[0, 0], conf[0, 0], loc[0, 0], npos


# speedup vs baseline: 1.9319x; 1.0662x over previous
import functools

import jax
import jax.numpy as jnp
from jax import lax
from jax.experimental import pallas as pl
from jax.experimental.pallas import tpu as pltpu

_B = 16
_N = 20000
_NOBJ = 20
_NCLS = 3
_SUB = 8
_LANE = 2500


def _fused_kernel(boxes_ref, labels_ref, priors_t_ref, locs_ref, scores_ref,
                  total_ref, conf_ref, loc_ref, npos_ref,
                  neg_sc, acc_sc, iou_sc, pri_sc):
    b = pl.program_id(0)

    @pl.when(b == 0)
    def _priors():
        pcx = priors_t_ref[0]
        pcy = priors_t_ref[1]
        pw = priors_t_ref[2]
        ph = priors_t_ref[3]
        x0 = pcx - pw / 2.0
        y0 = pcy - ph / 2.0
        x1 = pcx + pw / 2.0
        y1 = pcy + ph / 2.0
        pri_sc[0] = x0
        pri_sc[1] = y0
        pri_sc[2] = x1
        pri_sc[3] = y1
        pri_sc[4] = (x1 - x0) * (y1 - y0)
        pri_sc[5] = pcx
        pri_sc[6] = pcy
        pri_sc[7] = pw / 10.0
        pri_sc[8] = ph / 10.0
        pri_sc[9] = pw
        pri_sc[10] = ph

    px0 = pri_sc[0]
    py0 = pri_sc[1]
    px1 = pri_sc[2]
    py1 = pri_sc[3]
    parea = pri_sc[4]

    p_idx = (lax.broadcasted_iota(jnp.int32, (_SUB, _LANE), 0) * _LANE
             + lax.broadcasted_iota(jnp.int32, (_SUB, _LANE), 1))

    best = None
    besti = None
    for j in range(_NOBJ):
        bx0 = boxes_ref[0, j, 0]
        by0 = boxes_ref[0, j, 1]
        bx1 = boxes_ref[0, j, 2]
        by1 = boxes_ref[0, j, 3]
        barea = (bx1 - bx0) * (by1 - by0)
        ltx = jnp.maximum(px0, bx0)
        lty = jnp.maximum(py0, by0)
        rbx = jnp.minimum(px1, bx1)
        rby = jnp.minimum(py1, by1)
        w = jnp.maximum(rbx - ltx, 0.0)
        h = jnp.maximum(rby - lty, 0.0)
        inter = w * h
        iou = inter / (barea + parea - inter)
        if j == 0:
            best = iou
            besti = jnp.zeros((_SUB, _LANE), jnp.int32)
        else:
            gt = iou > best
            besti = jnp.where(gt, j, besti)
            best = jnp.where(gt, iou, best)
        iou_sc[j] = iou

    iou_all = iou_sc[...]
    mx = jnp.max(iou_all, axis=(1, 2), keepdims=True)
    pidx3 = p_idx[None]
    pj = jnp.min(jnp.where(iou_all == mx, pidx3, _N), axis=(1, 2),
                 keepdims=True)

    jv = lax.broadcasted_iota(jnp.int32, (_NOBJ, 1, 1), 0)
    jmax = jnp.max(jnp.where(pidx3 == pj, jv, -1), axis=0)
    hit = jmax >= 0
    besti = jnp.where(hit, jmax, besti)
    best = jnp.where(hit, 1.0, best)

    pa = labels_ref[0, 0, 0]
    pb = labels_ref[0, 0, 10]
    for jj in range(1, 10):
        pa = pa | (labels_ref[0, 0, jj] << (2 * jj))
        pb = pb | (labels_ref[0, 0, 10 + jj] << (2 * jj))
    sh = besti + besti
    lab_lo = jnp.right_shift(pa, sh) & 3
    lab_hi = jnp.right_shift(pb, jnp.maximum(sh - 20, 0)) & 3
    lab = jnp.where(besti < 10, lab_lo, lab_hi).astype(jnp.float32)

    bcx = jnp.zeros((_SUB, _LANE), jnp.float32)
    bcy = jnp.zeros((_SUB, _LANE), jnp.float32)
    bw = jnp.zeros((_SUB, _LANE), jnp.float32)
    bh = jnp.zeros((_SUB, _LANE), jnp.float32)
    for j in range(_NOBJ):
        m = besti == j
        jx0 = boxes_ref[0, j, 0]
        jy0 = boxes_ref[0, j, 1]
        jx1 = boxes_ref[0, j, 2]
        jy1 = boxes_ref[0, j, 3]
        bcx = jnp.where(m, (jx0 + jx1) / 2.0, bcx)
        bcy = jnp.where(m, (jy0 + jy1) / 2.0, bcy)
        bw = jnp.where(m, jx1 - jx0, bw)
        bh = jnp.where(m, jy1 - jy0, bh)

    lab = jnp.where(best < 0.5, 0.0, lab)
    posf = (lab > 0.0).astype(jnp.float32)
    n_pos = jnp.sum(posf)

    g0 = (bcx - pri_sc[5]) / pri_sc[7]
    g1 = (bcy - pri_sc[6]) / pri_sc[8]
    g2 = jnp.log(bw / pri_sc[9]) * 5.0
    g3 = jnp.log(bh / pri_sc[10]) * 5.0
    loc_sum = jnp.sum((jnp.abs(locs_ref[0, 0] - g0) + jnp.abs(locs_ref[0, 1] - g1)
                       + jnp.abs(locs_ref[0, 2] - g2) + jnp.abs(locs_ref[0, 3] - g3))
                      * posf)

    t_pair = (lab == 3.0).astype(jnp.float32)
    tgt0 = (lab == 0.0).astype(jnp.float32)
    tgt1 = (lab == 1.0).astype(jnp.float32) + t_pair
    tgt2 = (lab == 2.0).astype(jnp.float32) + t_pair
    bce = jnp.zeros((_SUB, _LANE), jnp.float32)
    for c, tgt in enumerate((tgt0, tgt1, tgt2)):
        l = scores_ref[0, c]
        bce = bce + (jnp.maximum(l, 0.0) - l * tgt + jnp.log1p(jnp.exp(-jnp.abs(l))))

    conf_pos = jnp.sum(bce * posf)
    neg_sc[b] = jnp.where(posf > 0.0, 0.0, bce)
    npos_ref[b] = n_pos.astype(jnp.int32)

    @pl.when(b == 0)
    def _init():
        acc_sc[0, 0] = loc_sum
        acc_sc[0, 1] = conf_pos

    @pl.when(b > 0)
    def _acc():
        acc_sc[0, 0] = acc_sc[0, 0] + loc_sum
        acc_sc[0, 1] = acc_sc[0, 1] + conf_pos

    @pl.when(b == _B - 1)
    def _epilogue():
        neg = neg_sc[...]
        bits = lax.bitcast_convert_type(neg, jnp.int32)
        img = lax.broadcasted_iota(jnp.int32, (_B, 1, 1), 0)
        nposv = jnp.zeros((_B, 1, 1), jnp.int32)
        for i in range(_B):
            nposv = jnp.where(img == i, npos_ref[i], nposv)
        n_pos_tot = jnp.sum(nposv).astype(jnp.float32)
        k = jnp.minimum(nposv * 3, _N)

        m = jnp.zeros((_B, 1, 1), jnp.int32)
        for bit in range(30, -1, -1):
            cand = m | (1 << bit)
            cnt = jnp.sum((bits >= cand).astype(jnp.int32), axis=(1, 2),
                          keepdims=True)
            m = jnp.where(cnt >= k, cand, m)
        tval = lax.bitcast_convert_type(m, jnp.float32)
        gtm = bits > m
        cnt_gt = jnp.sum(gtm.astype(jnp.float32), axis=(1, 2), keepdims=True)
        sum_gt = jnp.sum(jnp.where(gtm, neg, 0.0), axis=(1, 2), keepdims=True)
        s = sum_gt + (k.astype(jnp.float32) - cnt_gt) * tval
        s = jnp.where(k == 0, 0.0, s)
        hard = jnp.sum(s)

        conf_loss = (hard + acc_sc[0, 1]) / (1e-10 + n_pos_tot)
        loc_loss = acc_sc[0, 0] / jnp.maximum(4.0 * n_pos_tot, 1.0)
        total_ref[0, 0] = conf_loss + loc_loss
        conf_ref[0, 0] = conf_loss
        loc_ref[0, 0] = loc_loss


@functools.partial(jax.jit, static_argnames=())
def kernel(predicted_locs, predicted_scores, boxes, labels, priors_cxcy):
    priors_t = priors_cxcy.T.reshape(4, _SUB, _LANE)
    labels3 = labels.astype(jnp.int32).reshape(_B, 1, _NOBJ)

    locs_t = predicted_locs.transpose(0, 2, 1).reshape(_B, 4, _SUB, _LANE)
    scores_t = predicted_scores.transpose(0, 2, 1).reshape(_B, _NCLS, _SUB, _LANE)

    total, conf, loc, npos = pl.pallas_call(
        _fused_kernel,
        grid=(_B,),
        in_specs=[
            pl.BlockSpec((1, _NOBJ, 4), lambda b: (b, 0, 0),
                         memory_space=pltpu.SMEM),
            pl.BlockSpec((1, 1, _NOBJ), lambda b: (b, 0, 0),
                         memory_space=pltpu.SMEM),
            pl.BlockSpec((4, _SUB, _LANE), lambda b: (0, 0, 0)),
            pl.BlockSpec((1, 4, _SUB, _LANE), lambda b: (b, 0, 0, 0)),
            pl.BlockSpec((1, _NCLS, _SUB, _LANE), lambda b: (b, 0, 0, 0)),
        ],
        out_specs=[
            pl.BlockSpec((1, 1), lambda b: (0, 0), memory_space=pltpu.SMEM),
            pl.BlockSpec((1, 1), lambda b: (0, 0), memory_space=pltpu.SMEM),
            pl.BlockSpec((1, 1), lambda b: (0, 0), memory_space=pltpu.SMEM),
            pl.BlockSpec((_B,), lambda b: (0,), memory_space=pltpu.SMEM),
        ],
        out_shape=[
            jax.ShapeDtypeStruct((1, 1), jnp.float32),
            jax.ShapeDtypeStruct((1, 1), jnp.float32),
            jax.ShapeDtypeStruct((1, 1), jnp.float32),
            jax.ShapeDtypeStruct((_B,), jnp.int32),
        ],
        scratch_shapes=[
            pltpu.VMEM((_B, _SUB, _LANE), jnp.float32),
            pltpu.SMEM((1, 2), jnp.float32),
            pltpu.VMEM((_NOBJ, _SUB, _LANE), jnp.float32),
            pltpu.VMEM((11, _SUB, _LANE), jnp.float32),
        ],
    )(boxes, labels3, priors_t, locs_t, scores_t)

    return total[0, 0], conf[0, 0], loc[0, 0], npos
